# baseline (speedup 1.0000x reference)
import functools

import jax
import jax.numpy as jnp
from jax import lax
from jax.experimental import pallas as pl
from jax.experimental.pallas import tpu as pltpu
from jax.experimental.pallas import tpu_sc as plsc

N = 10000
E = 320000
D = 128
L = 3
K = 3
NEG_SLOPE = 0.01

NC = 2
NS = 16
NW = NC * NS
CHUNK = 128
NCH = 80
EPW = NCH * CHUNK
EPAD = EPW * NW
NBODY = NCH // 4
NPAD16 = ((N + 15) // 16) * 16
ROWS_PER_SUB = 624
ROW_BLK = 2000

_mesh = plsc.VectorSubcoreMesh(core_axis_name="c", subcore_axis_name="s")
_sc_params = pltpu.CompilerParams(needs_layout_passes=False)


@functools.partial(
    pl.kernel,
    mesh=_mesh,
    out_type=jax.ShapeDtypeStruct((NW * NPAD16,), jnp.float32),
    scratch_types=[
        pltpu.VMEM((NPAD16,), jnp.float32),
        pltpu.VMEM((EPW,), jnp.int32),
        pltpu.VMEM((EPW,), jnp.float32),
    ],
    compiler_params=_sc_params,
)
def _deg_kernel(col_hbm, w_hbm, out_hbm, deg_v, col_v, w_v):
    cid = lax.axis_index("c")
    sid = lax.axis_index("s")
    ew = cid * NS + sid
    zeros = jnp.zeros((16,), jnp.float32)

    def zb(i, carry):
        deg_v[pl.ds(i * 16, 16)] = zeros
        return carry

    lax.fori_loop(0, NPAD16 // 16, zb, None)
    pltpu.sync_copy(col_hbm.at[pl.ds(ew * EPW, EPW)], col_v)
    pltpu.sync_copy(w_hbm.at[pl.ds(ew * EPW, EPW)], w_v)

    def body(g, carry):
        cvec = col_v[pl.ds(g * 16, 16)]
        wvec = w_v[pl.ds(g * 16, 16)]
        plsc.addupdate_scatter(deg_v, [cvec], wvec)
        return carry

    lax.fori_loop(0, EPW // 16, body, None)
    pltpu.sync_copy(deg_v, out_hbm.at[pl.ds(ew * NPAD16, NPAD16)])


SUP = 8
NSUP = NCH // SUP


@functools.partial(
    pl.kernel,
    mesh=_mesh,
    out_type=jax.ShapeDtypeStruct((NC * N, D), jnp.float32),
    scratch_types=[
        pltpu.VMEM_SHARED((N, D), jnp.float32),
        pltpu.VMEM((CHUNK, D), jnp.float32),
        pltpu.VMEM((CHUNK, D), jnp.float32),
        pltpu.VMEM((3 * SUP, CHUNK), jnp.int32),
        pltpu.VMEM((3 * SUP, CHUNK), jnp.int32),
        pltpu.SemaphoreType.DMA,
        pltpu.SemaphoreType.DMA,
        pltpu.SemaphoreType.DMA,
        pltpu.SemaphoreType.DMA,
        pltpu.SemaphoreType.DMA,
        pltpu.SemaphoreType.DMA,
    ],
    compiler_params=_sc_params,
)
def _prop_kernel(u_hbm, ecomb_hbm, z_hbm, out_hbm,
                 acc_sh, buf0, buf1, stga, stgb,
                 gs0, gs1, ss0, ss1, sga, sgb):
    cid = lax.axis_index("c")
    sid = lax.axis_index("s")
    ew = cid * NS + sid
    ebase = ew * NCH * 3
    pltpu.sync_copy(z_hbm, acc_sh.at[pl.ds(sid * ROWS_PER_SUB, ROWS_PER_SUB)])

    @pl.when(sid == NS - 1)
    def _zero_tail():
        pltpu.sync_copy(z_hbm.at[pl.ds(0, 16)],
                        acc_sh.at[pl.ds(NS * ROWS_PER_SUB, 16)])

    plsc.subcore_barrier()
    lanes = lax.iota(jnp.int32, 16)
    bufs = (buf0, buf1)
    gsems = (gs0, gs1)
    ssems = (ss0, ss1)

    def scale(buf, stg, j):
        def cbody(c0, carry):
            wv = plsc.bitcast(stg[3 * j + 2, pl.ds(c0 * 16, 16)], jnp.float32)
            cvec = lanes + c0 * 16

            def fbody(f, carry2):
                fvec = jnp.full((16,), f, jnp.int32)
                vals = plsc.load_gather(buf, [cvec, fvec])
                plsc.store_scatter(buf, [cvec, fvec], vals * wv)
                return carry2

            lax.fori_loop(0, D, fbody, None, unroll=16)
            return carry

        lax.fori_loop(0, CHUNK // 16, cbody, None)

    def gwait(p):
        pltpu.make_async_copy(u_hbm.at[stga.at[0]], bufs[p], gsems[p]).wait()

    def swait(p):
        pltpu.make_async_copy(bufs[p], acc_sh.at[stga.at[1]], ssems[p]).wait()

    pltpu.sync_copy(ecomb_hbm.at[pl.ds(ebase, 3 * SUP)], stga)
    pltpu.async_copy(u_hbm.at[stga.at[0]], bufs[0], gsems[0])

    def one_chunk(t, sup_local, stg, stg_next, j, first, last):
        p = j % 2
        q = 1 - p
        gwait(p)
        scale(bufs[p], stg, j)
        if not first:
            swait(q)
        else:
            @pl.when(t > 0)
            def _():
                swait(q)
        if j < SUP - 1:
            pltpu.async_copy(u_hbm.at[stg.at[3 * (j + 1)]], bufs[q], gsems[q])
        elif not last:
            pltpu.async_copy(u_hbm.at[stg_next.at[0]], bufs[q], gsems[q])
        else:
            @pl.when(t < NSUP // 2 - 1)
            def _():
                pltpu.async_copy(u_hbm.at[stg_next.at[0]], bufs[q], gsems[q])
        pltpu.async_copy(bufs[p], acc_sh.at[stg.at[3 * j + 1]],
                         ssems[p], add=True)

    def body(t, carry):
        sbase = ebase + t * 2 * SUP * 3
        for j in range(SUP):
            if j == SUP - 1:
                pltpu.make_async_copy(
                    ecomb_hbm.at[pl.ds(sbase, 3 * SUP)], stgb, sgb).wait()
            one_chunk(t, 0, stga, stgb, j, first=(j == 0), last=False)
            if j == 0:
                pltpu.async_copy(
                    ecomb_hbm.at[pl.ds(sbase + 3 * SUP, 3 * SUP)], stgb, sgb)
        for j in range(SUP):
            if j == SUP - 1:
                @pl.when(t < NSUP // 2 - 1)
                def _():
                    pltpu.make_async_copy(
                        ecomb_hbm.at[pl.ds(sbase, 3 * SUP)], stga, sga).wait()
            one_chunk(t, 1, stgb, stga, j, first=False,
                      last=(j == SUP - 1))
            if j == 0:
                @pl.when(t < NSUP // 2 - 1)
                def _():
                    pltpu.async_copy(
                        ecomb_hbm.at[pl.ds(sbase + 6 * SUP, 3 * SUP)],
                        stga, sga)
        return carry

    lax.fori_loop(0, NSUP // 2, body, None)
    swait(1)
    plsc.subcore_barrier()
    pltpu.sync_copy(
        acc_sh.at[pl.ds(sid * ROWS_PER_SUB, ROWS_PER_SUB)],
        out_hbm.at[pl.ds(cid * N + sid * ROWS_PER_SUB, ROWS_PER_SUB)],
    )

    @pl.when(sid == NS - 1)
    def _out_tail():
        pltpu.sync_copy(
            acc_sh.at[pl.ds(NS * ROWS_PER_SUB, 16)],
            out_hbm.at[pl.ds(cid * N + NS * ROWS_PER_SUB, 16)],
        )


def _prep_body(parts_ref, x_ref, dis_ref, u0_ref):
    deg = jnp.sum(parts_ref[...], axis=0)
    dis = jnp.where(deg > 0, lax.rsqrt(jnp.where(deg > 0, deg, 1.0)), 0.0)
    dis_ref[...] = dis[:, None]
    u0_ref[...] = x_ref[...] * dis[:, None]


_prep = pl.pallas_call(
    _prep_body,
    out_shape=[
        jax.ShapeDtypeStruct((N, 1), jnp.float32),
        jax.ShapeDtypeStruct((N, D), jnp.float32),
    ],
)


def _comb_body(p_ref, dis_ref, h_ref, u_ref):
    s = p_ref[0] + p_ref[1]
    dis = dis_ref[...]
    h = s * dis
    h_ref[...] = h
    u_ref[...] = h * dis


_dspec = pl.BlockSpec((ROW_BLK, 1), lambda i: (i, 0))
_comb = pl.pallas_call(
    _comb_body,
    grid=(N // ROW_BLK,),
    in_specs=[
        pl.BlockSpec((NC, ROW_BLK, D), lambda i: (0, i, 0)),
        _dspec,
    ],
    out_specs=[
        pl.BlockSpec((ROW_BLK, D), lambda i: (i, 0)),
        pl.BlockSpec((ROW_BLK, D), lambda i: (i, 0)),
    ],
    out_shape=[
        jax.ShapeDtypeStruct((N, D), jnp.float32),
        jax.ShapeDtypeStruct((N, D), jnp.float32),
    ],
)


def _acc4(h_refs, w_ref, b_ref):
    acc = b_ref[...][None, :].astype(jnp.float32)
    for k in range(K + 1):
        acc = acc + jnp.dot(h_refs[k][...], w_ref[k],
                            preferred_element_type=jnp.float32)
    return acc


def _mid_body(h0, h1, h2, h3, w_ref, b_ref, dis_ref, h_ref, u_ref):
    acc = _acc4((h0, h1, h2, h3), w_ref, b_ref)
    h = jnp.where(acc > 0, acc, NEG_SLOPE * acc)
    h_ref[...] = h
    u_ref[...] = h * dis_ref[...]


_hspec = pl.BlockSpec((ROW_BLK, D), lambda i: (i, 0))
_mid = pl.pallas_call(
    _mid_body,
    grid=(N // ROW_BLK,),
    in_specs=[_hspec, _hspec, _hspec, _hspec,
              pl.BlockSpec((K + 1, D, D), lambda i: (0, 0, 0)),
              pl.BlockSpec((D,), lambda i: (0,)),
              _dspec],
    out_specs=[_hspec, _hspec],
    out_shape=[
        jax.ShapeDtypeStruct((N, D), jnp.float32),
        jax.ShapeDtypeStruct((N, D), jnp.float32),
    ],
)


def _last_body(h0, h1, h2, h3, w_ref, b_ref, mask_ref, o_ref):
    acc = _acc4((h0, h1, h2, h3), w_ref, b_ref)
    o_ref[...] = acc * mask_ref[...]


_last = pl.pallas_call(
    _last_body,
    grid=(N // ROW_BLK,),
    in_specs=[_hspec, _hspec, _hspec, _hspec,
              pl.BlockSpec((K + 1, D, D), lambda i: (0, 0, 0)),
              pl.BlockSpec((D,), lambda i: (0,)),
              _dspec],
    out_specs=_hspec,
    out_shape=jax.ShapeDtypeStruct((N, D), jnp.float32),
)


def kernel(x, edge_index, edge_weights, feature_mask, W, b):
    pad = EPAD - E
    row_p = jnp.concatenate(
        [edge_index[0], jnp.zeros((pad,), jnp.int32)]).reshape(NW * NCH, CHUNK)
    col_p = jnp.concatenate(
        [edge_index[1], jnp.zeros((pad,), jnp.int32)]).reshape(NW * NCH, CHUNK)
    w_p = jnp.concatenate([edge_weights, jnp.zeros((pad,), jnp.float32)])
    wbits = lax.bitcast_convert_type(w_p, jnp.int32).reshape(NW * NCH, CHUNK)
    ecomb = jnp.stack([row_p, col_p, wbits], axis=1).reshape(NW * NCH * 3,
                                                             CHUNK)
    z625 = jnp.zeros((ROWS_PER_SUB, D), jnp.float32)

    parts_deg = _deg_kernel(col_p.reshape(EPAD), w_p).reshape(NW, NPAD16)[:, :N]
    dis, u = _prep(parts_deg, x)

    h = x
    out = None
    for l in range(L):
        hs = [h]
        ucur = u
        for _ in range(K):
            part = _prop_kernel(ucur, ecomb, z625)
            hk, ucur = _comb(part.reshape(NC, N, D), dis)
            hs.append(hk)
        if l < L - 1:
            h, u = _mid(hs[0], hs[1], hs[2], hs[3], W[l], b[l], dis)
        else:
            out = _last(hs[0], hs[1], hs[2], hs[3], W[l], b[l],
                        feature_mask[:, None])
    return out


# speedup vs baseline: 2.3346x; 2.3346x over previous
import functools

import jax
import jax.numpy as jnp
from jax import lax
from jax.experimental import pallas as pl
from jax.experimental.pallas import tpu as pltpu
from jax.experimental.pallas import tpu_sc as plsc

N = 10000
E = 320000
D = 128
L = 3
K = 3
NEG_SLOPE = 0.01

NC = 2
NS = 16
NW = NC * NS
CHUNK = 128
NCH = 80
EPW = NCH * CHUNK
EPAD = EPW * NW
NBODY = NCH // 4
NPAD16 = ((N + 15) // 16) * 16
ROWS_PER_SUB = 624
ROW_BLK = 2000

_mesh = plsc.VectorSubcoreMesh(core_axis_name="c", subcore_axis_name="s")
_sc_params = pltpu.CompilerParams(needs_layout_passes=False)


@functools.partial(
    pl.kernel,
    mesh=_mesh,
    out_type=jax.ShapeDtypeStruct((NW * NPAD16,), jnp.float32),
    scratch_types=[
        pltpu.VMEM((NPAD16,), jnp.float32),
        pltpu.VMEM((EPW,), jnp.int32),
        pltpu.VMEM((EPW,), jnp.float32),
    ],
    compiler_params=_sc_params,
)
def _deg_kernel(col_hbm, w_hbm, out_hbm, deg_v, col_v, w_v):
    cid = lax.axis_index("c")
    sid = lax.axis_index("s")
    ew = cid * NS + sid
    zeros = jnp.zeros((16,), jnp.float32)

    def zb(i, carry):
        deg_v[pl.ds(i * 16, 16)] = zeros
        return carry

    lax.fori_loop(0, NPAD16 // 16, zb, None)
    pltpu.sync_copy(col_hbm.at[pl.ds(ew * EPW, EPW)], col_v)
    pltpu.sync_copy(w_hbm.at[pl.ds(ew * EPW, EPW)], w_v)

    def body(g, carry):
        cvec = col_v[pl.ds(g * 16, 16)]
        wvec = w_v[pl.ds(g * 16, 16)]
        plsc.addupdate_scatter(deg_v, [cvec], wvec)
        return carry

    lax.fori_loop(0, EPW // 16, body, None)
    pltpu.sync_copy(deg_v, out_hbm.at[pl.ds(ew * NPAD16, NPAD16)])


SUP = 8
NSUP = NCH // SUP


@functools.partial(
    pl.kernel,
    mesh=_mesh,
    out_type=jax.ShapeDtypeStruct((NC * N, D), jnp.float32),
    scratch_types=[
        pltpu.VMEM_SHARED((N, D), jnp.float32),
        pltpu.VMEM((CHUNK, D), jnp.float32),
        pltpu.VMEM((CHUNK, D), jnp.float32),
        pltpu.VMEM((3 * SUP, CHUNK), jnp.int32),
        pltpu.VMEM((3 * SUP, CHUNK), jnp.int32),
        pltpu.SemaphoreType.DMA,
        pltpu.SemaphoreType.DMA,
        pltpu.SemaphoreType.DMA,
        pltpu.SemaphoreType.DMA,
        pltpu.SemaphoreType.DMA,
        pltpu.SemaphoreType.DMA,
    ],
    compiler_params=_sc_params,
)
def _prop_kernel(u_hbm, ecomb_hbm, z_hbm, out_hbm,
                 acc_sh, buf0, buf1, stga, stgb,
                 gs0, gs1, ss0, ss1, sga, sgb):
    cid = lax.axis_index("c")
    sid = lax.axis_index("s")
    ew = cid * NS + sid
    ebase = ew * NCH * 3
    pltpu.sync_copy(z_hbm, acc_sh.at[pl.ds(sid * ROWS_PER_SUB, ROWS_PER_SUB)])

    @pl.when(sid == NS - 1)
    def _zero_tail():
        pltpu.sync_copy(z_hbm.at[pl.ds(0, 16)],
                        acc_sh.at[pl.ds(NS * ROWS_PER_SUB, 16)])

    plsc.subcore_barrier()
    lanes = lax.iota(jnp.int32, 16)
    bufs = (buf0, buf1)
    gsems = (gs0, gs1)
    ssems = (ss0, ss1)

    def scale(buf, stg, j):
        def cbody(c0, carry):
            wv = plsc.bitcast(stg[3 * j + 2, pl.ds(c0 * 16, 16)], jnp.float32)
            cvec = lanes + c0 * 16

            def fbody(f, carry2):
                fvec = (jnp.full((16,), f, jnp.int32) + lanes) & (D - 1)
                vals = plsc.load_gather(buf, [cvec, fvec])
                plsc.store_scatter(buf, [cvec, fvec], vals * wv)
                return carry2

            lax.fori_loop(0, D, fbody, None, unroll=16)
            return carry

        lax.fori_loop(0, CHUNK // 16, cbody, None)

    def gwait(p):
        pltpu.make_async_copy(u_hbm.at[stga.at[0]], bufs[p], gsems[p]).wait()

    def swait(p):
        pltpu.make_async_copy(bufs[p], acc_sh.at[stga.at[1]], ssems[p]).wait()

    pltpu.sync_copy(ecomb_hbm.at[pl.ds(ebase, 3 * SUP)], stga)
    pltpu.async_copy(u_hbm.at[stga.at[0]], bufs[0], gsems[0])

    def one_chunk(t, sup_local, stg, stg_next, j, first, last):
        p = j % 2
        q = 1 - p
        gwait(p)
        scale(bufs[p], stg, j)
        if not first:
            swait(q)
        else:
            @pl.when(t > 0)
            def _():
                swait(q)
        if j < SUP - 1:
            pltpu.async_copy(u_hbm.at[stg.at[3 * (j + 1)]], bufs[q], gsems[q])
        elif not last:
            pltpu.async_copy(u_hbm.at[stg_next.at[0]], bufs[q], gsems[q])
        else:
            @pl.when(t < NSUP // 2 - 1)
            def _():
                pltpu.async_copy(u_hbm.at[stg_next.at[0]], bufs[q], gsems[q])
        pltpu.async_copy(bufs[p], acc_sh.at[stg.at[3 * j + 1]],
                         ssems[p], add=True)

    def body(t, carry):
        sbase = ebase + t * 2 * SUP * 3
        for j in range(SUP):
            if j == SUP - 1:
                pltpu.make_async_copy(
                    ecomb_hbm.at[pl.ds(sbase, 3 * SUP)], stgb, sgb).wait()
            one_chunk(t, 0, stga, stgb, j, first=(j == 0), last=False)
            if j == 0:
                pltpu.async_copy(
                    ecomb_hbm.at[pl.ds(sbase + 3 * SUP, 3 * SUP)], stgb, sgb)
        for j in range(SUP):
            if j == SUP - 1:
                @pl.when(t < NSUP // 2 - 1)
                def _():
                    pltpu.make_async_copy(
                        ecomb_hbm.at[pl.ds(sbase, 3 * SUP)], stga, sga).wait()
            one_chunk(t, 1, stgb, stga, j, first=False,
                      last=(j == SUP - 1))
            if j == 0:
                @pl.when(t < NSUP // 2 - 1)
                def _():
                    pltpu.async_copy(
                        ecomb_hbm.at[pl.ds(sbase + 6 * SUP, 3 * SUP)],
                        stga, sga)
        return carry

    lax.fori_loop(0, NSUP // 2, body, None)
    swait(1)
    plsc.subcore_barrier()
    pltpu.sync_copy(
        acc_sh.at[pl.ds(sid * ROWS_PER_SUB, ROWS_PER_SUB)],
        out_hbm.at[pl.ds(cid * N + sid * ROWS_PER_SUB, ROWS_PER_SUB)],
    )

    @pl.when(sid == NS - 1)
    def _out_tail():
        pltpu.sync_copy(
            acc_sh.at[pl.ds(NS * ROWS_PER_SUB, 16)],
            out_hbm.at[pl.ds(cid * N + NS * ROWS_PER_SUB, 16)],
        )


def _prep_body(parts_ref, x_ref, dis_ref, u0_ref):
    deg = jnp.sum(parts_ref[...], axis=0)
    dis = jnp.where(deg > 0, lax.rsqrt(jnp.where(deg > 0, deg, 1.0)), 0.0)
    dis_ref[...] = dis[:, None]
    u0_ref[...] = x_ref[...] * dis[:, None]


_prep = pl.pallas_call(
    _prep_body,
    out_shape=[
        jax.ShapeDtypeStruct((N, 1), jnp.float32),
        jax.ShapeDtypeStruct((N, D), jnp.float32),
    ],
)


def _comb_body(p_ref, dis_ref, h_ref, u_ref):
    s = p_ref[0] + p_ref[1]
    dis = dis_ref[...]
    h = s * dis
    h_ref[...] = h
    u_ref[...] = h * dis


_dspec = pl.BlockSpec((ROW_BLK, 1), lambda i: (i, 0))
_comb = pl.pallas_call(
    _comb_body,
    grid=(N // ROW_BLK,),
    in_specs=[
        pl.BlockSpec((NC, ROW_BLK, D), lambda i: (0, i, 0)),
        _dspec,
    ],
    out_specs=[
        pl.BlockSpec((ROW_BLK, D), lambda i: (i, 0)),
        pl.BlockSpec((ROW_BLK, D), lambda i: (i, 0)),
    ],
    out_shape=[
        jax.ShapeDtypeStruct((N, D), jnp.float32),
        jax.ShapeDtypeStruct((N, D), jnp.float32),
    ],
)


def _acc4(h_refs, w_ref, b_ref):
    acc = b_ref[...][None, :].astype(jnp.float32)
    for k in range(K + 1):
        acc = acc + jnp.dot(h_refs[k][...], w_ref[k],
                            preferred_element_type=jnp.float32)
    return acc


def _mid_body(h0, h1, h2, h3, w_ref, b_ref, dis_ref, h_ref, u_ref):
    acc = _acc4((h0, h1, h2, h3), w_ref, b_ref)
    h = jnp.where(acc > 0, acc, NEG_SLOPE * acc)
    h_ref[...] = h
    u_ref[...] = h * dis_ref[...]


_hspec = pl.BlockSpec((ROW_BLK, D), lambda i: (i, 0))
_mid = pl.pallas_call(
    _mid_body,
    grid=(N // ROW_BLK,),
    in_specs=[_hspec, _hspec, _hspec, _hspec,
              pl.BlockSpec((K + 1, D, D), lambda i: (0, 0, 0)),
              pl.BlockSpec((D,), lambda i: (0,)),
              _dspec],
    out_specs=[_hspec, _hspec],
    out_shape=[
        jax.ShapeDtypeStruct((N, D), jnp.float32),
        jax.ShapeDtypeStruct((N, D), jnp.float32),
    ],
)


def _last_body(h0, h1, h2, h3, w_ref, b_ref, mask_ref, o_ref):
    acc = _acc4((h0, h1, h2, h3), w_ref, b_ref)
    o_ref[...] = acc * mask_ref[...]


_last = pl.pallas_call(
    _last_body,
    grid=(N // ROW_BLK,),
    in_specs=[_hspec, _hspec, _hspec, _hspec,
              pl.BlockSpec((K + 1, D, D), lambda i: (0, 0, 0)),
              pl.BlockSpec((D,), lambda i: (0,)),
              _dspec],
    out_specs=_hspec,
    out_shape=jax.ShapeDtypeStruct((N, D), jnp.float32),
)


def kernel(x, edge_index, edge_weights, feature_mask, W, b):
    pad = EPAD - E
    row_p = jnp.concatenate(
        [edge_index[0], jnp.zeros((pad,), jnp.int32)]).reshape(NW * NCH, CHUNK)
    col_p = jnp.concatenate(
        [edge_index[1], jnp.zeros((pad,), jnp.int32)]).reshape(NW * NCH, CHUNK)
    w_p = jnp.concatenate([edge_weights, jnp.zeros((pad,), jnp.float32)])
    wbits = lax.bitcast_convert_type(w_p, jnp.int32).reshape(NW * NCH, CHUNK)
    ecomb = jnp.stack([row_p, col_p, wbits], axis=1).reshape(NW * NCH * 3,
                                                             CHUNK)
    z625 = jnp.zeros((ROWS_PER_SUB, D), jnp.float32)

    parts_deg = _deg_kernel(col_p.reshape(EPAD), w_p).reshape(NW, NPAD16)[:, :N]
    dis, u = _prep(parts_deg, x)

    h = x
    out = None
    for l in range(L):
        hs = [h]
        ucur = u
        for _ in range(K):
            part = _prop_kernel(ucur, ecomb, z625)
            hk, ucur = _comb(part.reshape(NC, N, D), dis)
            hs.append(hk)
        if l < L - 1:
            h, u = _mid(hs[0], hs[1], hs[2], hs[3], W[l], b[l], dis)
        else:
            out = _last(hs[0], hs[1], hs[2], hs[3], W[l], b[l],
                        feature_mask[:, None])
    return out
